# baseline (speedup 1.0000x reference)
import functools

import jax
import jax.numpy as jnp
from jax import lax
from jax.experimental import pallas as pl
from jax.experimental.pallas import tpu as pltpu
from jax.experimental.pallas import tpu_sc as plsc

B = 16384
F = 32
L = 16
NC = 2
NS = 16
NW = NC * NS

SC_ROWS = 6144
TC_ROWS = B - SC_ROWS
ROWS_PER_W = SC_ROWS // NW
GROUP = 8
NGROUPS = ROWS_PER_W // GROUP
TC_BLOCK = 2048

_SPLAT_DNUMS = lax.GatherDimensionNumbers(
    offset_dims=(), collapsed_slice_dims=(0,), start_index_map=(0,))


def _splat(vec, lane):
    idx = jnp.full((L, 1), lane, dtype=jnp.int32)
    return lax.gather(vec, idx, _SPLAT_DNUMS, (1,),
                      mode=lax.GatherScatterMode.PROMISE_IN_BOUNDS)


@functools.partial(
    pl.kernel,
    out_type=jax.ShapeDtypeStruct((SC_ROWS, F), jnp.float32),
    mesh=plsc.VectorSubcoreMesh(core_axis_name="c", subcore_axis_name="s"),
    compiler_params=pltpu.CompilerParams(use_tc_tiling_on_sc=True),
    scratch_types=[
        pltpu.VMEM((ROWS_PER_W, F), jnp.float32),
        pltpu.VMEM((F, F), jnp.float32),
        pltpu.VMEM((F,), jnp.float32),
        pltpu.SemaphoreType.DMA,
        pltpu.SemaphoreType.DMA,
        pltpu.SemaphoreType.DMA,
    ],
)
def _sc_linear(din_hbm, w_hbm, b_hbm, out_hbm, x_v, w_v, b_v, s0, s1, s2):
    wid = lax.axis_index("s") * NC + lax.axis_index("c")
    base = wid * ROWS_PER_W

    cx = pltpu.async_copy(din_hbm.at[pl.ds(base, ROWS_PER_W), :], x_v, s0)
    cw = pltpu.async_copy(w_hbm, w_v, s1)
    cb = pltpu.async_copy(b_hbm, b_v, s2)
    cw.wait()
    cb.wait()

    b_lo = b_v[pl.ds(0, L)]
    b_hi = b_v[pl.ds(L, L)]
    bv0 = jnp.zeros((L,), jnp.float32)
    bv1 = jnp.zeros((L,), jnp.float32)
    for j in range(F):
        s = _splat(b_lo if j < L else b_hi, j % L)
        bv0 = bv0 + s * w_v[j, pl.ds(0, L)]
        bv1 = bv1 + s * w_v[j, pl.ds(L, L)]
    cx.wait()

    def group_body(g, carry):
        bv0, bv1 = carry
        r0 = g * GROUP
        xs = []
        for r in range(GROUP):
            xs.append((x_v[r0 + r, pl.ds(0, L)], x_v[r0 + r, pl.ds(L, L)]))
        acc = [(bv0, bv1)] * GROUP
        for j in range(F):
            w0 = w_v[j, pl.ds(0, L)]
            w1 = w_v[j, pl.ds(L, L)]
            half, lane = divmod(j, L)
            for r in range(GROUP):
                s = _splat(xs[r][half], lane)
                a0, a1 = acc[r]
                acc[r] = (a0 + s * w0, a1 + s * w1)
        zero = jnp.zeros((L,), jnp.float32)
        for r in range(GROUP):
            a0, a1 = acc[r]
            x_v[r0 + r, pl.ds(0, L)] = jnp.maximum(a0, zero)
            x_v[r0 + r, pl.ds(L, L)] = jnp.maximum(a1, zero)
        return bv0, bv1

    lax.fori_loop(0, NGROUPS, group_body, (bv0, bv1))

    pltpu.sync_copy(x_v, out_hbm.at[pl.ds(base, ROWS_PER_W), :])


def _tc_body(x_ref, w_ref, b_ref, o_ref):
    y = jnp.dot(x_ref[...] + b_ref[...], w_ref[...],
                preferred_element_type=jnp.float32,
                precision=lax.Precision.HIGHEST)
    o_ref[...] = jnp.maximum(y, 0.0)


def _tc_linear(din, weight, bias):
    return pl.pallas_call(
        _tc_body,
        grid=(TC_ROWS // TC_BLOCK,),
        in_specs=[
            pl.BlockSpec((TC_BLOCK, F), lambda i: (i + SC_ROWS // TC_BLOCK, 0)),
            pl.BlockSpec((F, F), lambda i: (0, 0)),
            pl.BlockSpec((1, F), lambda i: (0, 0)),
        ],
        out_specs=pl.BlockSpec((TC_BLOCK, F), lambda i: (i, 0)),
        out_shape=jax.ShapeDtypeStruct((TC_ROWS, F), jnp.float32),
    )(din, weight, bias.reshape(1, F))


def kernel(din, weight, bias):
    out_sc = _sc_linear(din[:SC_ROWS], weight, bias)
    out_tc = _tc_linear(din, weight, bias)
    return jnp.concatenate([out_sc, out_tc], axis=0)


# speedup vs baseline: 1.0961x; 1.0961x over previous
import functools

import jax
import jax.numpy as jnp
from jax import lax
from jax.experimental import pallas as pl
from jax.experimental.pallas import tpu as pltpu
from jax.experimental.pallas import tpu_sc as plsc

B = 16384
F = 32
L = 16
NC = 2
NS = 16
NW = NC * NS

SC_ROWS = 6144
TC_ROWS = B - SC_ROWS
ROWS_PER_W = SC_ROWS // NW
GROUP = 8
NGROUPS = ROWS_PER_W // GROUP
TC_BLOCK = 2048

_SPLAT_DNUMS = lax.GatherDimensionNumbers(
    offset_dims=(), collapsed_slice_dims=(0,), start_index_map=(0,))


def _splat(vec, lane):
    idx = jnp.full((L, 1), lane, dtype=jnp.int32)
    return lax.gather(vec, idx, _SPLAT_DNUMS, (1,),
                      mode=lax.GatherScatterMode.PROMISE_IN_BOUNDS)


@functools.partial(
    pl.kernel,
    out_type=jax.ShapeDtypeStruct((SC_ROWS, F), jnp.float32),
    mesh=plsc.VectorSubcoreMesh(core_axis_name="c", subcore_axis_name="s"),
    compiler_params=pltpu.CompilerParams(use_tc_tiling_on_sc=True),
    scratch_types=[
        pltpu.VMEM((ROWS_PER_W, F), jnp.float32),
        pltpu.VMEM((F, F), jnp.float32),
        pltpu.VMEM((F,), jnp.float32),
        pltpu.SemaphoreType.DMA,
        pltpu.SemaphoreType.DMA,
        pltpu.SemaphoreType.DMA,
    ],
)
def _sc_linear(din_hbm, w_hbm, b_hbm, out_hbm, x_v, w_v, b_v, s0, s1, s2):
    wid = lax.axis_index("s") * NC + lax.axis_index("c")
    base = wid * ROWS_PER_W

    cx = pltpu.async_copy(din_hbm.at[pl.ds(base, ROWS_PER_W), :], x_v, s0)
    cw = pltpu.async_copy(w_hbm, w_v, s1)
    cb = pltpu.async_copy(b_hbm, b_v, s2)
    cw.wait()
    cb.wait()

    b_lo = b_v[pl.ds(0, L)]
    b_hi = b_v[pl.ds(L, L)]
    bv0 = jnp.zeros((L,), jnp.float32)
    bv1 = jnp.zeros((L,), jnp.float32)
    for j in range(F):
        s = _splat(b_lo if j < L else b_hi, j % L)
        bv0 = bv0 + s * w_v[j, pl.ds(0, L)]
        bv1 = bv1 + s * w_v[j, pl.ds(L, L)]
    cx.wait()

    def group_body(g, carry):
        bv0, bv1 = carry
        r0 = g * GROUP
        xs = []
        for r in range(GROUP):
            xs.append((x_v[r0 + r, pl.ds(0, L)], x_v[r0 + r, pl.ds(L, L)]))
        acc = [(bv0, bv1)] * GROUP
        for j in range(F):
            w0 = w_v[j, pl.ds(0, L)]
            w1 = w_v[j, pl.ds(L, L)]
            half, lane = divmod(j, L)
            for r in range(GROUP):
                s = _splat(xs[r][half], lane)
                a0, a1 = acc[r]
                acc[r] = (a0 + s * w0, a1 + s * w1)
        zero = jnp.zeros((L,), jnp.float32)
        for r in range(GROUP):
            a0, a1 = acc[r]
            x_v[r0 + r, pl.ds(0, L)] = jnp.maximum(a0, zero)
            x_v[r0 + r, pl.ds(L, L)] = jnp.maximum(a1, zero)
        return bv0, bv1

    lax.fori_loop(0, NGROUPS, group_body, (bv0, bv1))

    pltpu.sync_copy(x_v, out_hbm.at[pl.ds(base, ROWS_PER_W), :])


def _tc_body(x_ref, w_ref, b_ref, o_ref):
    y = jnp.dot(x_ref[...] + b_ref[...], w_ref[...],
                preferred_element_type=jnp.float32,
                precision=lax.Precision.HIGHEST)
    o_ref[...] = jnp.maximum(y, 0.0)


def _tc_linear(din, weight, bias):
    return pl.pallas_call(
        _tc_body,
        grid=(TC_ROWS // TC_BLOCK,),
        in_specs=[
            pl.BlockSpec((TC_BLOCK, F), lambda i: (i + SC_ROWS // TC_BLOCK, 0)),
            pl.BlockSpec((F, F), lambda i: (0, 0)),
            pl.BlockSpec((1, F), lambda i: (0, 0)),
        ],
        out_specs=pl.BlockSpec((TC_BLOCK, F), lambda i: (i, 0)),
        out_shape=jax.ShapeDtypeStruct((TC_ROWS, F), jnp.float32),
    )(din, weight, bias.reshape(1, F))


def kernel(din, weight, bias):
    out_sc = _sc_linear(din, weight, bias)
    out_tc = _tc_linear(din, weight, bias)
    return jnp.concatenate([out_sc, out_tc], axis=0)
